# baseline (speedup 1.0000x reference)
import functools

import jax
import jax.numpy as jnp
from jax import lax
from jax.experimental import pallas as pl
from jax.experimental.pallas import tpu as pltpu

N_REC = 3
TOK_BLOCK = 1024


def _tc_body(x_ref, w_ref, logits_ref, gate_ref):
    xb = x_ref[...]
    wb = w_ref[...]
    logits = lax.dot_general(
        xb, wb, (((1,), (1,)), ((), ())),
        preferred_element_type=jnp.float32)
    m = jnp.max(logits, axis=-1, keepdims=True)
    s = jnp.sum(jnp.exp(logits - m), axis=-1, keepdims=True)
    logits_ref[...] = logits
    gate_ref[...] = 1.0 / s


def kernel(x, W):
    B, T, D = x.shape
    N = B * T
    xf = x.reshape(N, D)
    grid = N // TOK_BLOCK
    logits, gate = pl.pallas_call(
        _tc_body,
        grid=(grid,),
        in_specs=[
            pl.BlockSpec((TOK_BLOCK, D), lambda i: (i, 0)),
            pl.BlockSpec((N_REC, D), lambda i: (0, 0)),
        ],
        out_specs=[
            pl.BlockSpec((TOK_BLOCK, N_REC), lambda i: (i, 0)),
            pl.BlockSpec((TOK_BLOCK, 1), lambda i: (i, 0)),
        ],
        out_shape=[
            jax.ShapeDtypeStruct((N, N_REC), jnp.float32),
            jax.ShapeDtypeStruct((N, 1), jnp.float32),
        ],
    )(xf, W)
    selected = jnp.broadcast_to(
        jnp.arange(T, dtype=jnp.int32)[None, :, None], (B, T, 1))
    return selected, gate.reshape(B, T, 1), logits.reshape(B, T, N_REC)
